# baseline (speedup 1.0000x reference)
import functools

import jax
import jax.numpy as jnp
from jax import lax
from jax.experimental import pallas as pl
from jax.experimental.pallas import tpu as pltpu
from jax.experimental.pallas import tpu_sc as plsc

BATCH = 16384
BIT = 64
L = 16
NC = 2
NS = 16
NW = NC * NS
B_PER_W = BATCH // NW
CHUNKS = BIT // L

_mesh = plsc.VectorSubcoreMesh(core_axis_name="c", subcore_axis_name="s")


@functools.partial(
    pl.kernel,
    out_type=jax.ShapeDtypeStruct((NW, L), jnp.float32),
    mesh=_mesh,
    scratch_types=[
        pltpu.VMEM((B_PER_W,), jnp.int32),
        pltpu.VMEM((B_PER_W, BIT), jnp.float32),
        pltpu.VMEM((B_PER_W, BIT), jnp.float32),
        pltpu.VMEM((L,), jnp.float32),
        pltpu.SemaphoreType.DMA,
    ],
    compiler_params=pltpu.CompilerParams(use_tc_tiling_on_sc=False),
)
def _mse_gather(u_hbm, ind_hbm, h_hbm, out_hbm, idx_v, rows_v, u_v, acc_v, sem):
    wid = lax.axis_index("s") * NC + lax.axis_index("c")
    base = wid * B_PER_W
    pltpu.sync_copy(ind_hbm.at[pl.ds(base, B_PER_W)], idx_v)
    gather = pltpu.async_copy(h_hbm.at[idx_v], rows_v, sem)
    pltpu.sync_copy(u_hbm.at[pl.ds(base, B_PER_W)], u_v)
    gather.wait()

    def row_body(i, accs):
        out = []
        for j in range(CHUNKS):
            d = u_v[i, pl.ds(j * L, L)] - rows_v[i, pl.ds(j * L, L)]
            out.append(accs[j] + d * d)
        return tuple(out)

    zero = jnp.zeros((L,), jnp.float32)
    a = lax.fori_loop(0, B_PER_W, row_body, (zero, zero, zero, zero))
    acc_v[...] = (a[0] + a[1]) + (a[2] + a[3])
    pltpu.sync_copy(acc_v, out_hbm.at[wid])


def kernel(u, y, ind, H):
    del y
    partials = _mse_gather(u, ind.astype(jnp.int32), H)
    return jnp.sum(partials) * (1.0 / (BATCH * BIT))


# speedup vs baseline: 1.8424x; 1.8424x over previous
import functools

import jax
import jax.numpy as jnp
from jax import lax
from jax.experimental import pallas as pl
from jax.experimental.pallas import tpu as pltpu
from jax.experimental.pallas import tpu_sc as plsc

BATCH = 16384
BIT = 64
L = 16
NC = 2
NS = 16
NW = NC * NS
COLS_PER_W = BIT // NW
NTRAIN = 100000
UCHUNK = 8192

_mesh = plsc.VectorSubcoreMesh(core_axis_name="c", subcore_axis_name="s")


@functools.partial(
    pl.kernel,
    out_type=jax.ShapeDtypeStruct((NW, L), jnp.float32),
    mesh=_mesh,
    scratch_types=[
        pltpu.VMEM((NTRAIN,), jnp.float32),
        pltpu.VMEM((BATCH,), jnp.int32),
        pltpu.VMEM((UCHUNK,), jnp.float32),
        pltpu.VMEM((L,), jnp.float32),
    ],
    compiler_params=pltpu.CompilerParams(needs_layout_passes=False),
)
def _mse_cols(ut_hbm, ind_hbm, ht_hbm, out_hbm, hcol_v, ind_v, ut_v, acc_v):
    wid = lax.axis_index("s") * NC + lax.axis_index("c")
    pltpu.sync_copy(ind_hbm, ind_v)

    acc = jnp.zeros((L,), jnp.float32)
    for t in range(COLS_PER_W):
        col = wid * COLS_PER_W + t
        pltpu.sync_copy(ht_hbm.at[col], hcol_v)
        for c in range(BATCH // UCHUNK):
            pltpu.sync_copy(ut_hbm.at[col, pl.ds(c * UCHUNK, UCHUNK)], ut_v)
            base = c * UCHUNK

            def body(k, a, base=base):
                idx = ind_v[pl.ds(base + k * L, L)]
                g = plsc.load_gather(hcol_v, [idx])
                d = ut_v[pl.ds(k * L, L)] - g
                return a + d * d

            acc = lax.fori_loop(0, UCHUNK // L, body, acc)
    acc_v[...] = acc
    pltpu.sync_copy(acc_v, out_hbm.at[wid])


def kernel(u, y, ind, H):
    del y
    partials = _mse_cols(u.T, ind.astype(jnp.int32), H.T)
    return jnp.sum(partials) * (1.0 / (BATCH * BIT))


# speedup vs baseline: 2.1599x; 1.1724x over previous
import functools

import jax
import jax.numpy as jnp
from jax import lax
from jax.experimental import pallas as pl
from jax.experimental.pallas import tpu as pltpu
from jax.experimental.pallas import tpu_sc as plsc

BATCH = 16384
BIT = 64
L = 16
NC = 2
NS = 16
NW = NC * NS
COLS_PER_W = BIT // NW
NTRAIN = 100000
UCHUNK = 8192

_mesh = plsc.VectorSubcoreMesh(core_axis_name="c", subcore_axis_name="s")


@functools.partial(
    pl.kernel,
    out_type=jax.ShapeDtypeStruct((NW, L), jnp.float32),
    mesh=_mesh,
    scratch_types=[
        pltpu.VMEM((NTRAIN,), jnp.float32),
        pltpu.VMEM((BATCH,), jnp.int32),
        pltpu.VMEM((UCHUNK,), jnp.float32),
        pltpu.VMEM((L,), jnp.float32),
    ],
    compiler_params=pltpu.CompilerParams(needs_layout_passes=False),
)
def _mse_cols(ut_hbm, ind_hbm, ht_hbm, out_hbm, hcol_v, ind_v, ut_v, acc_v):
    wid = lax.axis_index("s") * NC + lax.axis_index("c")
    pltpu.sync_copy(ind_hbm, ind_v)

    zero = jnp.zeros((L,), jnp.float32)
    accs = (zero, zero, zero, zero)
    for t in range(COLS_PER_W):
        col = wid * COLS_PER_W + t
        pltpu.sync_copy(ht_hbm.at[col], hcol_v)
        for c in range(BATCH // UCHUNK):
            pltpu.sync_copy(ut_hbm.at[col, pl.ds(c * UCHUNK, UCHUNK)], ut_v)
            cbase = c * UCHUNK

            @plsc.parallel_loop(0, UCHUNK, 4 * L, unroll=2, carry=accs)
            def body(k, a, cbase=cbase):
                out = []
                for q in range(4):
                    idx = ind_v[pl.ds(cbase + k + q * L, L)]
                    g = plsc.load_gather(hcol_v, [idx])
                    d = ut_v[pl.ds(k + q * L, L)] - g
                    out.append(a[q] + d * d)
                return tuple(out)

            accs = body
    acc_v[...] = (accs[0] + accs[1]) + (accs[2] + accs[3])
    pltpu.sync_copy(acc_v, out_hbm.at[wid])


def kernel(u, y, ind, H):
    del y
    partials = _mse_cols(u.T, ind.astype(jnp.int32), H.T)
    return jnp.sum(partials) * (1.0 / (BATCH * BIT))
